# baseline (speedup 1.0000x reference)
import functools

import jax
import jax.numpy as jnp
from jax import lax
from jax.experimental import pallas as pl
from jax.experimental.pallas import tpu as pltpu
from jax.experimental.pallas import tpu_sc as plsc


def _sc_pq_lookup(ids_flat, codes_tbl, cent2d):
    n_tokens = ids_flat.shape[0]
    icb = codes_tbl.shape[1] // 2
    vals = 256
    sub = cent2d.shape[1]

    info = plsc.get_sparse_core_info()
    nw = info.num_cores * info.num_subcores
    per_tile = n_tokens // nw
    nb = 128
    n_iters = per_tile // nb
    ch = 128
    rows_per_blk = nb * icb
    nch = rows_per_blk // ch

    mesh = plsc.VectorSubcoreMesh(core_axis_name="c", subcore_axis_name="s")

    @functools.partial(
        pl.kernel,
        out_type=jax.ShapeDtypeStruct((n_tokens * icb, sub), jnp.float32),
        mesh=mesh,
        scratch_types=[
            pltpu.VMEM((per_tile,), jnp.int32),
            pltpu.VMEM((2, nb, 2 * icb), jnp.int32),
            pltpu.VMEM((2, rows_per_blk), jnp.int32),
            pltpu.VMEM((2, rows_per_blk, sub), jnp.float32),
            pltpu.VMEM_SHARED((vals * icb, sub), jnp.float32),
            [pltpu.SemaphoreType.DMA] * 2,
            [pltpu.SemaphoreType.DMA] * 2,
            [pltpu.SemaphoreType.DMA] * 2,
        ],
        compiler_params=pltpu.CompilerParams(use_tc_tiling_on_sc=False),
    )
    def k(ids_hbm, codes_hbm, cent_hbm, out_hbm,
          ids_v, cw_v, r_v, rows_v, cent_s, sem_c, sem_r, sem_s):
        sid = lax.axis_index("s")
        wid = sid * info.num_cores + lax.axis_index("c")
        base = wid * per_tile

        @pl.when(sid == 0)
        def _():
            pltpu.sync_copy(cent_hbm, cent_s)

        pltpu.sync_copy(ids_hbm.at[pl.ds(base, per_tile)], ids_v)
        plsc.subcore_barrier()

        i16 = lax.iota(jnp.int32, 16)
        joff = lax.bitwise_and(i16, icb - 1) * vals
        lo_half = i16 < icb

        def fire_codes(t, p):
            pltpu.async_copy(
                codes_hbm.at[ids_v.at[pl.ds(t * nb, nb)]],
                cw_v.at[p], sem_c[p])

        def wait_codes(p):
            pltpu.make_async_copy(
                codes_hbm.at[pl.ds(0, nb), :], cw_v.at[p], sem_c[p]).wait()

        def compute_r(p):
            for v in range(rows_per_blk // 16):
                c0 = cw_v[p, 2 * v, :]
                c1 = cw_v[p, 2 * v + 1, :]
                r_v[p, pl.ds(v * 16, 16)] = jnp.where(lo_half, c0, c1) + joff

        def fire_rows(p):
            for rc in range(nch):
                pltpu.async_copy(
                    cent_s.at[r_v.at[p, pl.ds(rc * ch, ch)]],
                    rows_v.at[p, pl.ds(rc * ch, ch), :], sem_r[p])

        def wait_rows(p):
            pltpu.make_async_copy(
                out_hbm.at[pl.ds(0, rows_per_blk), :],
                rows_v.at[p], sem_r[p]).wait()

        def fire_scatter(t, p):
            return pltpu.async_copy(
                rows_v.at[p],
                out_hbm.at[pl.ds((base + t * nb) * icb, rows_per_blk), :],
                sem_s[p])

        def wait_scatter(p):
            pltpu.make_async_copy(
                rows_v.at[p],
                out_hbm.at[pl.ds(0, rows_per_blk), :], sem_s[p]).wait()

        fire_codes(0, 0)
        wait_codes(0)
        fire_codes(1, 1)
        compute_r(0)
        fire_rows(0)

        @pl.loop(0, (n_iters - 2) // 2)
        def _(i):
            for d in range(2):
                t = 2 * i + 1 + d
                p = (1 + d) % 2
                wait_codes(p)
                compute_r(p)
                fire_codes(t + 1, 1 - p)

                @pl.when(t >= 2)
                def _():
                    wait_scatter(p)

                fire_rows(p)
                wait_rows(1 - p)
                fire_scatter(t - 1, 1 - p)

        t_last = n_iters - 1
        wait_codes(1)
        compute_r(1)
        wait_scatter(1)
        fire_rows(1)
        wait_rows(0)
        fire_scatter(t_last - 1, 0)
        wait_rows(1)
        fire_scatter(t_last, 1)
        wait_scatter(0)
        wait_scatter(1)

    return k(ids_flat, codes_tbl, cent2d)


def kernel(input_ids, batch_size, item_codes, centroids):
    bsz, seq = input_ids.shape
    icb, vals, sub = centroids.shape
    n_items = item_codes.shape[0]
    ids_flat = input_ids.reshape(bsz * seq)
    codes_i32 = item_codes.astype(jnp.int32)
    k = jnp.arange(n_items * 2 * icb, dtype=jnp.int32)
    codes_wide = codes_i32[k >> 4, k & (icb - 1)].reshape(n_items, 2 * icb)
    cent2d = centroids.reshape(icb * vals, sub)
    out = _sc_pq_lookup(ids_flat, codes_wide, cent2d)
    return out.reshape(bsz, seq, icb * sub)


# speedup vs baseline: 95.4078x; 95.4078x over previous
import functools

import jax
import jax.numpy as jnp
from jax import lax
from jax.experimental import pallas as pl
from jax.experimental.pallas import tpu as pltpu
from jax.experimental.pallas import tpu_sc as plsc


def _sc_pq_lookup(ids_flat, codes_tbl, cent2d):
    n_tokens = ids_flat.shape[0]
    icb = codes_tbl.shape[1] // 2
    vals = 256
    sub = cent2d.shape[1]

    info = plsc.get_sparse_core_info()
    nw = info.num_cores * info.num_subcores
    per_tile = n_tokens // nw
    nb = 128
    n_iters = per_tile // nb
    ch = 128
    rows_per_blk = nb * icb
    nch = rows_per_blk // ch

    mesh = plsc.VectorSubcoreMesh(core_axis_name="c", subcore_axis_name="s")

    @functools.partial(
        pl.kernel,
        out_type=jax.ShapeDtypeStruct((n_tokens * icb, sub), jnp.float32),
        mesh=mesh,
        scratch_types=[
            pltpu.VMEM((per_tile,), jnp.int32),
            pltpu.VMEM((2, nb, 2 * icb), jnp.int32),
            pltpu.VMEM((2, rows_per_blk), jnp.int32),
            pltpu.VMEM((2, rows_per_blk, sub), jnp.float32),
            pltpu.VMEM_SHARED((vals * icb, sub), jnp.float32),
            [pltpu.SemaphoreType.DMA] * 2,
            [pltpu.SemaphoreType.DMA] * 2,
            [pltpu.SemaphoreType.DMA] * 2,
        ],
        compiler_params=pltpu.CompilerParams(use_tc_tiling_on_sc=False),
    )
    def k(ids_hbm, codes_hbm, cent_hbm, out_hbm,
          ids_v, cw_v, r_v, rows_v, cent_s, sem_c, sem_r, sem_s):
        sid = lax.axis_index("s")
        wid = sid * info.num_cores + lax.axis_index("c")
        base = wid * per_tile

        @pl.when(sid == 0)
        def _():
            pltpu.sync_copy(cent_hbm, cent_s)

        pltpu.sync_copy(ids_hbm.at[pl.ds(base, per_tile)], ids_v)
        plsc.subcore_barrier()

        i16 = lax.iota(jnp.int32, 16)
        joff = lax.bitwise_and(i16, icb - 1) * vals
        lo_half = i16 < icb

        def fire_codes(t, p):
            pltpu.async_copy(
                codes_hbm.at[ids_v.at[pl.ds(t * nb, nb)]],
                cw_v.at[p], sem_c[p])

        def wait_codes(p):
            pltpu.make_async_copy(
                codes_hbm.at[pl.ds(0, nb), :], cw_v.at[p], sem_c[p]).wait()

        def compute_r(p):
            for v in range(rows_per_blk // 16):
                c0 = cw_v[p, 2 * v, :]
                c1 = cw_v[p, 2 * v + 1, :]
                r_v[p, pl.ds(v * 16, 16)] = jnp.where(lo_half, c0, c1) + joff

        def fire_rows(p):
            for rc in range(nch):
                pltpu.async_copy(
                    cent_s.at[r_v.at[p, pl.ds(rc * ch, ch)]],
                    rows_v.at[p, pl.ds(rc * ch, ch), :], sem_r[p])

        def wait_rows(p):
            pltpu.make_async_copy(
                out_hbm.at[pl.ds(0, rows_per_blk), :],
                rows_v.at[p], sem_r[p]).wait()

        def fire_scatter(t, p):
            return pltpu.async_copy(
                rows_v.at[p],
                out_hbm.at[pl.ds((base + t * nb) * icb, rows_per_blk), :],
                sem_s[p])

        def wait_scatter(p):
            pltpu.make_async_copy(
                rows_v.at[p],
                out_hbm.at[pl.ds(0, rows_per_blk), :], sem_s[p]).wait()

        fire_codes(0, 0)
        wait_codes(0)
        fire_codes(1, 1)
        compute_r(0)
        fire_rows(0)

        @pl.loop(0, (n_iters - 2) // 2)
        def _(i):
            for d in range(2):
                t = 2 * i + 1 + d
                p = (1 + d) % 2
                wait_codes(p)
                compute_r(p)
                fire_codes(t + 1, 1 - p)

                @pl.when(t >= 2)
                def _():
                    wait_scatter(p)

                fire_rows(p)
                wait_rows(1 - p)
                fire_scatter(t - 1, 1 - p)

        t_last = n_iters - 1
        wait_codes(1)
        compute_r(1)
        wait_scatter(1)
        fire_rows(1)
        wait_rows(0)
        fire_scatter(t_last - 1, 0)
        wait_rows(1)
        fire_scatter(t_last, 1)
        wait_scatter(0)
        wait_scatter(1)

    return k(ids_flat, codes_tbl, cent2d)


def kernel(input_ids, batch_size, item_codes, centroids):
    bsz, seq = input_ids.shape
    icb, vals, sub = centroids.shape
    n_items = item_codes.shape[0]
    ids_flat = input_ids.reshape(bsz * seq)
    codes_i32 = item_codes.astype(jnp.int32)
    codes_wide = jnp.broadcast_to(
        codes_i32[:, None, :], (n_items, 2, icb)
    ).reshape(n_items * 2 * icb).reshape(n_items, 2 * icb)
    cent2d = centroids.reshape(icb * vals, sub)
    out = _sc_pq_lookup(ids_flat, codes_wide, cent2d)
    return out.reshape(bsz, seq, icb * sub)
